# baseline (speedup 1.0000x reference)
import functools

import jax
import jax.numpy as jnp
from jax import lax
from jax.experimental import pallas as pl
from jax.experimental.pallas import tpu as pltpu
from jax.experimental.pallas import tpu_sc as plsc

_BN = 1000
_CH = 128
_NTILE = 16
_NCORE = 2
_EPS = 1e-6


def _layer_norm_in_kernel(h, g, b):
    mu = jnp.mean(h, axis=-1, keepdims=True)
    var = jnp.mean((h - mu) * (h - mu), axis=-1, keepdims=True)
    return (h - mu) / jnp.sqrt(var + _EPS) * g + b


def _embed_ln_msgsrc_body(vocab, hs, n, x_ref, emb_ref, g_ref, b_ref,
                          wcat_ref, col_ref, attr_ref, hn_ref, xws_ref,
                          gidx_ref):
    xb = x_ref[0, 0, :]
    oh = (xb[:, None] == lax.broadcasted_iota(jnp.int32, (xb.shape[0], vocab), 1)
          ).astype(jnp.float32)
    h = jnp.dot(oh, emb_ref[...], preferred_element_type=jnp.float32)
    hn = _layer_norm_in_kernel(h, g_ref[0, :], b_ref[0, :])
    hn_ref[...] = hn
    xw = jnp.dot(hn, wcat_ref[...], preferred_element_type=jnp.float32)
    for t in range(4):
        xws_ref[t] = xw[:, t * hs:(t + 1) * hs]
    gidx_ref[0] = attr_ref[0] * n + col_ref[0]


def _mid_body(hs, hn_ref, parts_ref, w1_ref, b1_ref, w2_ref, b2_ref,
              g_ref, b_ref, wcat_ref, hn2_ref, xws_ref):
    h = hn_ref[...] + parts_ref[0] + parts_ref[1]
    inter = jnp.dot(h, w1_ref[...], preferred_element_type=jnp.float32) + b1_ref[0, :]
    inter = jnp.where(inter >= 0, inter, 0.01 * inter)
    h = h + jnp.dot(inter, w2_ref[...], preferred_element_type=jnp.float32) + b2_ref[0, :]
    hn = _layer_norm_in_kernel(h, g_ref[0, :], b_ref[0, :])
    hn2_ref[...] = hn
    xw = jnp.dot(hn, wcat_ref[...], preferred_element_type=jnp.float32)
    for t in range(4):
        xws_ref[t] = xw[:, t * hs:(t + 1) * hs]


def _final_body(nblocks, ngraphs, hn_ref, parts_ref, w1_ref, b1_ref, w2_ref,
                b2_ref, batch_ref, hw_ref, hb_ref, hf_ref, vec_ref, prop_ref,
                cnt_ref):
    i = pl.program_id(0)
    h = hn_ref[...] + parts_ref[0] + parts_ref[1]
    inter = jnp.dot(h, w1_ref[...], preferred_element_type=jnp.float32) + b1_ref[0, :]
    inter = jnp.where(inter >= 0, inter, 0.01 * inter)
    hf = h + jnp.dot(inter, w2_ref[...], preferred_element_type=jnp.float32) + b2_ref[0, :]
    hf_ref[...] = hf

    bb = batch_ref[0, 0, :]
    oh = (bb[:, None] == lax.broadcasted_iota(jnp.int32, (bb.shape[0], ngraphs), 1)
          ).astype(jnp.float32)
    psum = lax.dot_general(oh, hf, (((0,), (0,)), ((), ())),
                           preferred_element_type=jnp.float32)
    pcnt = jnp.sum(oh, axis=0)

    @pl.when(i == 0)
    def _():
        vec_ref[...] = psum
        cnt_ref[0, :] = pcnt

    @pl.when(i > 0)
    def _():
        vec_ref[...] += psum
        cnt_ref[0, :] += pcnt

    @pl.when(i == nblocks - 1)
    def _():
        cnt = jnp.maximum(cnt_ref[0, :], 1.0)
        v = vec_ref[...] / cnt[:, None]
        vec_ref[...] = v
        prop_ref[...] = (jnp.sum(v * hw_ref[...], axis=1, keepdims=True)
                         + hb_ref[0, 0])


_SB = 8


def _sc_edge_scatter(xwflat, gidx2, row2, n_nodes, hs):
    nch = gidx2.shape[0]
    nworkers = _NCORE * _NTILE
    assert nch % (nworkers * _SB) == 0
    nsuper = nch // (nworkers * _SB)
    per_w = nsuper * _SB

    n_acc = n_nodes + 8
    unit = (n_nodes // _NTILE) // 8 * 8
    tail = n_nodes - _NTILE * unit
    nfull = unit // _CH
    rem = unit - nfull * _CH

    mesh = plsc.VectorSubcoreMesh(core_axis_name="c", subcore_axis_name="s")

    @functools.partial(
        pl.kernel,
        out_type=jax.ShapeDtypeStruct((_NCORE, n_nodes, hs), jnp.float32),
        mesh=mesh,
        scratch_types=[
            pltpu.VMEM_SHARED((n_acc, hs), jnp.float32),
            pltpu.VMEM((_SB, _CH), jnp.int32),
            pltpu.VMEM((_SB, _CH), jnp.int32),
            pltpu.VMEM((_CH, hs), jnp.float32),
            pltpu.VMEM((_CH, hs), jnp.float32),
            pltpu.SemaphoreType.DMA,
            pltpu.SemaphoreType.DMA,
            pltpu.SemaphoreType.DMA,
            pltpu.SemaphoreType.DMA,
        ],
    )
    def k(xw_hbm, gidx_hbm, rowi_hbm, out_hbm,
          acc_sh, gidx_sb, row_sb, rows0, rows1, gsem0, gsem1, ssem0, ssem1):
        cid = lax.axis_index("c")
        sid = lax.axis_index("s")
        wid = cid * _NTILE + sid

        zeros16 = jnp.zeros((16,), jnp.float32)

        def zrow(r, _):
            for j in range(hs // 16):
                rows0[r, pl.ds(j * 16, 16)] = zeros16
            return 0

        lax.fori_loop(0, _CH, zrow, 0)
        r0 = sid * unit
        for kk in range(nfull):
            pltpu.sync_copy(rows0, acc_sh.at[pl.ds(r0 + kk * _CH, _CH)])
        if rem > 0:
            pltpu.sync_copy(rows0.at[pl.ds(0, rem)],
                            acc_sh.at[pl.ds(r0 + nfull * _CH, rem)])

        @pl.when(sid == _NTILE - 1)
        def _():
            pltpu.sync_copy(rows0.at[pl.ds(0, tail + 8)],
                            acc_sh.at[pl.ds(_NTILE * unit, tail + 8)])

        plsc.subcore_barrier()

        c_base = wid * per_w

        def super_chunk(s, _):
            c0 = c_base + s * _SB
            pltpu.sync_copy(gidx_hbm.at[pl.ds(c0, _SB)], gidx_sb)
            pltpu.sync_copy(rowi_hbm.at[pl.ds(c0, _SB)], row_sb)
            bufs = (rows0, rows1)
            gsems = (gsem0, gsem1)
            ssems = (ssem0, ssem1)
            gd = [None] * _SB
            sd = [None] * _SB
            for kk in range(_SB):
                b = kk % 2
                if kk >= 2:
                    sd[kk - 2].wait()
                gd[kk] = pltpu.async_copy(xw_hbm.at[gidx_sb.at[kk]],
                                          bufs[b], gsems[b])
                if kk >= 1:
                    pb = (kk - 1) % 2
                    gd[kk - 1].wait()
                    sd[kk - 1] = pltpu.async_copy(
                        bufs[pb], acc_sh.at[row_sb.at[kk - 1]], ssems[pb],
                        add=True)
            last = _SB - 1
            gd[last].wait()
            sd[last] = pltpu.async_copy(
                bufs[last % 2], acc_sh.at[row_sb.at[last]], ssems[last % 2],
                add=True)
            sd[last - 1].wait()
            sd[last].wait()
            return 0

        lax.fori_loop(0, nsuper, super_chunk, 0)
        plsc.subcore_barrier()

        for kk in range(nfull):
            pltpu.sync_copy(acc_sh.at[pl.ds(r0 + kk * _CH, _CH)],
                            out_hbm.at[cid, pl.ds(r0 + kk * _CH, _CH)])
        if rem > 0:
            pltpu.sync_copy(acc_sh.at[pl.ds(r0 + nfull * _CH, rem)],
                            out_hbm.at[cid, pl.ds(r0 + nfull * _CH, rem)])

        @pl.when(sid == _NTILE - 1)
        def _():
            if tail > 0:
                pltpu.sync_copy(acc_sh.at[pl.ds(_NTILE * unit, tail)],
                                out_hbm.at[cid, pl.ds(_NTILE * unit, tail)])

    return k(xwflat, gidx2, row2)


def kernel(x, edge_index, edge_attr, batch, params):
    n = x.shape[0]
    hs = params['emb'].shape[1]
    vocab = params['emb'].shape[0]
    nint = params['ff_w1_0'].shape[1]
    ngraphs = 64
    nb = n // _BN
    assert n % _BN == 0

    row = edge_index[0].astype(jnp.int32)
    col = edge_index[1].astype(jnp.int32)
    attr = edge_attr.astype(jnp.int32)
    x3 = x.astype(jnp.int32).reshape(nb, 1, _BN)
    batch3 = batch.astype(jnp.int32).reshape(nb, 1, _BN)

    ne = row.shape[0]
    assert ne % _CH == 0
    nch = ne // _CH
    gran = _NCORE * _NTILE * _SB
    nch_pad = -(-nch // gran) * gran
    row2 = jnp.concatenate(
        [row.reshape(nch, _CH),
         jnp.full((nch_pad - nch, _CH), n, jnp.int32)], axis=0)

    f32 = jnp.float32
    row_spec = pl.BlockSpec((_BN, hs), lambda i: (i, 0))
    idx_spec = pl.BlockSpec((1, 1, _BN), lambda i: (i, 0, 0))
    parts_spec = pl.BlockSpec((_NCORE, _BN, hs), lambda i: (0, i, 0))
    xws_spec = pl.BlockSpec((4, _BN, hs), lambda i: (0, i, 0))
    full = lambda shape: pl.BlockSpec(shape, lambda i: tuple(0 for _ in shape))

    def wcat(l):
        names = ['i', 'ii', 'iii', 'a']
        return jnp.concatenate([params[f'W_{nm}_{l}'] for nm in names], axis=1)

    ech = nch // nb
    assert nch % nb == 0
    ech_spec = pl.BlockSpec((1, ech, _CH), lambda i: (i, 0, 0))
    hn0, xws0, gidx3 = pl.pallas_call(
        functools.partial(_embed_ln_msgsrc_body, vocab, hs, n),
        grid=(nb,),
        in_specs=[idx_spec, full((vocab, hs)), full((1, hs)), full((1, hs)),
                  full((hs, 4 * hs)), ech_spec, ech_spec],
        out_specs=[row_spec, xws_spec, ech_spec],
        out_shape=[jax.ShapeDtypeStruct((n, hs), f32),
                   jax.ShapeDtypeStruct((4, n, hs), f32),
                   jax.ShapeDtypeStruct((nb, ech, _CH), jnp.int32)],
    )(x3, params['emb'], params['ln_g_0'].reshape(1, hs),
      params['ln_b_0'].reshape(1, hs), wcat(0),
      col.reshape(nb, ech, _CH), attr.reshape(nb, ech, _CH))

    gidx2 = jnp.concatenate(
        [gidx3.reshape(nch, _CH),
         jnp.zeros((nch_pad - nch, _CH), jnp.int32)], axis=0)

    parts0 = _sc_edge_scatter(xws0.reshape(4 * n, hs), gidx2, row2, n, hs)

    hn1, xws1 = pl.pallas_call(
        functools.partial(_mid_body, hs),
        grid=(nb,),
        in_specs=[row_spec, parts_spec, full((hs, nint)), full((1, nint)),
                  full((nint, hs)), full((1, hs)), full((1, hs)),
                  full((1, hs)), full((hs, 4 * hs))],
        out_specs=[row_spec, xws_spec],
        out_shape=[jax.ShapeDtypeStruct((n, hs), f32),
                   jax.ShapeDtypeStruct((4, n, hs), f32)],
    )(hn0, parts0, params['ff_w1_0'], params['ff_b1_0'].reshape(1, -1),
      params['ff_w2_0'], params['ff_b2_0'].reshape(1, hs),
      params['ln_g_1'].reshape(1, hs), params['ln_b_1'].reshape(1, hs),
      wcat(1))

    parts1 = _sc_edge_scatter(xws1.reshape(4 * n, hs), gidx2, row2, n, hs)

    hw_row = params['head_w'].reshape(hs)[None, :]
    hf, vec, prop = pl.pallas_call(
        functools.partial(_final_body, nb, ngraphs),
        grid=(nb,),
        in_specs=[row_spec, parts_spec, full((hs, nint)), full((1, nint)),
                  full((nint, hs)), full((1, hs)), idx_spec,
                  full((1, hs)), full((1, 1))],
        out_specs=[row_spec,
                   pl.BlockSpec((ngraphs, hs), lambda i: (0, 0)),
                   pl.BlockSpec((ngraphs, 1), lambda i: (0, 0))],
        out_shape=[jax.ShapeDtypeStruct((n, hs), f32),
                   jax.ShapeDtypeStruct((ngraphs, hs), f32),
                   jax.ShapeDtypeStruct((ngraphs, 1), f32)],
        scratch_shapes=[pltpu.VMEM((1, ngraphs), f32)],
    )(hn1, parts1, params['ff_w1_1'], params['ff_b1_1'].reshape(1, -1),
      params['ff_w2_1'], params['ff_b2_1'].reshape(1, hs), batch3,
      hw_row, params['head_b'].reshape(1, 1))

    return prop, vec, hf


# speedup vs baseline: 3.3004x; 3.3004x over previous
import functools

import jax
import jax.numpy as jnp
from jax import lax
from jax.experimental import pallas as pl
from jax.experimental.pallas import tpu as pltpu
from jax.experimental.pallas import tpu_sc as plsc

_BN = 1000
_CH = 128
_NTILE = 16
_NCORE = 2
_EPS = 1e-6


def _layer_norm_in_kernel(h, g, b):
    mu = jnp.mean(h, axis=-1, keepdims=True)
    var = jnp.mean((h - mu) * (h - mu), axis=-1, keepdims=True)
    return (h - mu) / jnp.sqrt(var + _EPS) * g + b


def _embed_ln_msgsrc_body(vocab, hs, n, x_ref, emb_ref, g_ref, b_ref,
                          wcat_ref, col_ref, attr_ref, hn_ref, xws_ref,
                          gidx_ref):
    xb = x_ref[0, 0, :]
    oh = (xb[:, None] == lax.broadcasted_iota(jnp.int32, (xb.shape[0], vocab), 1)
          ).astype(jnp.float32)
    h = jnp.dot(oh, emb_ref[...], preferred_element_type=jnp.float32)
    hn = _layer_norm_in_kernel(h, g_ref[0, :], b_ref[0, :])
    hn_ref[...] = hn
    xw = jnp.dot(hn, wcat_ref[...], preferred_element_type=jnp.float32)
    for t in range(4):
        xws_ref[t] = xw[:, t * hs:(t + 1) * hs]
    gidx_ref[0] = attr_ref[0] * n + col_ref[0]


def _mid_body(hs, hn_ref, parts_ref, w1_ref, b1_ref, w2_ref, b2_ref,
              g_ref, b_ref, wcat_ref, hn2_ref, xws_ref):
    h = hn_ref[...] + parts_ref[0] + parts_ref[1]
    inter = jnp.dot(h, w1_ref[...], preferred_element_type=jnp.float32) + b1_ref[0, :]
    inter = jnp.where(inter >= 0, inter, 0.01 * inter)
    h = h + jnp.dot(inter, w2_ref[...], preferred_element_type=jnp.float32) + b2_ref[0, :]
    hn = _layer_norm_in_kernel(h, g_ref[0, :], b_ref[0, :])
    hn2_ref[...] = hn
    xw = jnp.dot(hn, wcat_ref[...], preferred_element_type=jnp.float32)
    for t in range(4):
        xws_ref[t] = xw[:, t * hs:(t + 1) * hs]


def _final_body(nblocks, ngraphs, hn_ref, parts_ref, w1_ref, b1_ref, w2_ref,
                b2_ref, batch_ref, hw_ref, hb_ref, hf_ref, vec_ref, prop_ref,
                cnt_ref):
    i = pl.program_id(0)
    h = hn_ref[...] + parts_ref[0] + parts_ref[1]
    inter = jnp.dot(h, w1_ref[...], preferred_element_type=jnp.float32) + b1_ref[0, :]
    inter = jnp.where(inter >= 0, inter, 0.01 * inter)
    hf = h + jnp.dot(inter, w2_ref[...], preferred_element_type=jnp.float32) + b2_ref[0, :]
    hf_ref[...] = hf

    bb = batch_ref[0, 0, :]
    oh = (bb[:, None] == lax.broadcasted_iota(jnp.int32, (bb.shape[0], ngraphs), 1)
          ).astype(jnp.float32)
    psum = lax.dot_general(oh, hf, (((0,), (0,)), ((), ())),
                           preferred_element_type=jnp.float32)
    pcnt = jnp.sum(oh, axis=0)

    @pl.when(i == 0)
    def _():
        vec_ref[...] = psum
        cnt_ref[0, :] = pcnt

    @pl.when(i > 0)
    def _():
        vec_ref[...] += psum
        cnt_ref[0, :] += pcnt

    @pl.when(i == nblocks - 1)
    def _():
        cnt = jnp.maximum(cnt_ref[0, :], 1.0)
        v = vec_ref[...] / cnt[:, None]
        vec_ref[...] = v
        prop_ref[...] = (jnp.sum(v * hw_ref[...], axis=1, keepdims=True)
                         + hb_ref[0, 0])


_SB = 8


def _sc_edge_scatter(xwflat, gidx2, row2, n_nodes, hs):
    nch = gidx2.shape[0]
    nworkers = _NCORE * _NTILE
    assert nch % (nworkers * _SB) == 0
    nsuper = nch // (nworkers * _SB)
    per_w = nsuper * _SB

    n_acc = n_nodes + _CH
    unit = (n_nodes // _NTILE) // 8 * 8
    tail = n_nodes - _NTILE * unit
    nfull = unit // _CH
    rem = unit - nfull * _CH

    mesh = plsc.VectorSubcoreMesh(core_axis_name="c", subcore_axis_name="s")

    @functools.partial(
        pl.kernel,
        out_type=jax.ShapeDtypeStruct((_NCORE, n_nodes, hs), jnp.float32),
        mesh=mesh,
        scratch_types=[
            pltpu.VMEM_SHARED((n_acc, hs), jnp.float32),
            pltpu.VMEM((_SB, _CH), jnp.int32),
            pltpu.VMEM((_SB, _CH), jnp.int32),
            pltpu.VMEM((_CH, hs), jnp.float32),
            pltpu.VMEM((_CH, hs), jnp.float32),
            pltpu.SemaphoreType.DMA,
            pltpu.SemaphoreType.DMA,
            pltpu.SemaphoreType.DMA,
            pltpu.SemaphoreType.DMA,
        ],
    )
    def k(xw_hbm, gidx_hbm, rowi_hbm, out_hbm,
          acc_sh, gidx_sb, row_sb, rows0, rows1, gsem0, gsem1, ssem0, ssem1):
        cid = lax.axis_index("c")
        sid = lax.axis_index("s")
        wid = cid * _NTILE + sid

        zeros16 = jnp.zeros((16,), jnp.float32)

        def zrow(r, _):
            for j in range(hs // 16):
                rows0[r, pl.ds(j * 16, 16)] = zeros16
            return 0

        lax.fori_loop(0, _CH, zrow, 0)
        r0 = sid * unit
        for kk in range(nfull):
            pltpu.sync_copy(rows0, acc_sh.at[pl.ds(r0 + kk * _CH, _CH)])
        if rem > 0:
            pltpu.sync_copy(rows0.at[pl.ds(0, rem)],
                            acc_sh.at[pl.ds(r0 + nfull * _CH, rem)])

        @pl.when(sid == _NTILE - 1)
        def _():
            if tail > 0:
                pltpu.sync_copy(rows0.at[pl.ds(0, tail)],
                                acc_sh.at[pl.ds(_NTILE * unit, tail)])

        plsc.subcore_barrier()

        c_base = wid * per_w

        def super_chunk(s, _):
            c0 = c_base + s * _SB
            pltpu.sync_copy(gidx_hbm.at[pl.ds(c0, _SB)], gidx_sb)
            pltpu.sync_copy(rowi_hbm.at[pl.ds(c0, _SB)], row_sb)
            bufs = (rows0, rows1)
            gsems = (gsem0, gsem1)
            ssems = (ssem0, ssem1)
            gd = [None] * _SB
            sd = [None] * _SB
            for kk in range(_SB):
                b = kk % 2
                if kk >= 2:
                    sd[kk - 2].wait()
                gd[kk] = pltpu.async_copy(xw_hbm.at[gidx_sb.at[kk]],
                                          bufs[b], gsems[b])
                if kk >= 1:
                    pb = (kk - 1) % 2
                    gd[kk - 1].wait()
                    sd[kk - 1] = pltpu.async_copy(
                        bufs[pb], acc_sh.at[row_sb.at[kk - 1]], ssems[pb],
                        add=True)
            last = _SB - 1
            gd[last].wait()
            sd[last] = pltpu.async_copy(
                bufs[last % 2], acc_sh.at[row_sb.at[last]], ssems[last % 2],
                add=True)
            sd[last - 1].wait()
            sd[last].wait()
            return 0

        lax.fori_loop(0, nsuper, super_chunk, 0)
        plsc.subcore_barrier()

        for kk in range(nfull):
            pltpu.sync_copy(acc_sh.at[pl.ds(r0 + kk * _CH, _CH)],
                            out_hbm.at[cid, pl.ds(r0 + kk * _CH, _CH)])
        if rem > 0:
            pltpu.sync_copy(acc_sh.at[pl.ds(r0 + nfull * _CH, rem)],
                            out_hbm.at[cid, pl.ds(r0 + nfull * _CH, rem)])

        @pl.when(sid == _NTILE - 1)
        def _():
            if tail > 0:
                pltpu.sync_copy(acc_sh.at[pl.ds(_NTILE * unit, tail)],
                                out_hbm.at[cid, pl.ds(_NTILE * unit, tail)])

    return k(xwflat, gidx2, row2)


def kernel(x, edge_index, edge_attr, batch, params):
    n = x.shape[0]
    hs = params['emb'].shape[1]
    vocab = params['emb'].shape[0]
    nint = params['ff_w1_0'].shape[1]
    ngraphs = 64
    nb = n // _BN
    assert n % _BN == 0

    row = edge_index[0].astype(jnp.int32)
    col = edge_index[1].astype(jnp.int32)
    attr = edge_attr.astype(jnp.int32)
    x3 = x.astype(jnp.int32).reshape(nb, 1, _BN)
    batch3 = batch.astype(jnp.int32).reshape(nb, 1, _BN)

    ne = row.shape[0]
    assert ne % _CH == 0
    nch = ne // _CH
    gran = _NCORE * _NTILE * _SB
    nch_pad = -(-nch // gran) * gran
    trash = n + jnp.arange(_CH, dtype=jnp.int32)
    row2 = jnp.concatenate(
        [row.reshape(nch, _CH),
         jnp.broadcast_to(trash, (nch_pad - nch, _CH))], axis=0)

    f32 = jnp.float32
    row_spec = pl.BlockSpec((_BN, hs), lambda i: (i, 0))
    idx_spec = pl.BlockSpec((1, 1, _BN), lambda i: (i, 0, 0))
    parts_spec = pl.BlockSpec((_NCORE, _BN, hs), lambda i: (0, i, 0))
    xws_spec = pl.BlockSpec((4, _BN, hs), lambda i: (0, i, 0))
    full = lambda shape: pl.BlockSpec(shape, lambda i: tuple(0 for _ in shape))

    def wcat(l):
        names = ['i', 'ii', 'iii', 'a']
        return jnp.concatenate([params[f'W_{nm}_{l}'] for nm in names], axis=1)

    ech = nch // nb
    assert nch % nb == 0
    ech_spec = pl.BlockSpec((1, ech, _CH), lambda i: (i, 0, 0))
    hn0, xws0, gidx3 = pl.pallas_call(
        functools.partial(_embed_ln_msgsrc_body, vocab, hs, n),
        grid=(nb,),
        in_specs=[idx_spec, full((vocab, hs)), full((1, hs)), full((1, hs)),
                  full((hs, 4 * hs)), ech_spec, ech_spec],
        out_specs=[row_spec, xws_spec, ech_spec],
        out_shape=[jax.ShapeDtypeStruct((n, hs), f32),
                   jax.ShapeDtypeStruct((4, n, hs), f32),
                   jax.ShapeDtypeStruct((nb, ech, _CH), jnp.int32)],
    )(x3, params['emb'], params['ln_g_0'].reshape(1, hs),
      params['ln_b_0'].reshape(1, hs), wcat(0),
      col.reshape(nb, ech, _CH), attr.reshape(nb, ech, _CH))

    gidx2 = jnp.concatenate(
        [gidx3.reshape(nch, _CH),
         jnp.broadcast_to(jnp.arange(_CH, dtype=jnp.int32),
                          (nch_pad - nch, _CH))], axis=0)

    parts0 = _sc_edge_scatter(xws0.reshape(4 * n, hs), gidx2, row2, n, hs)

    hn1, xws1 = pl.pallas_call(
        functools.partial(_mid_body, hs),
        grid=(nb,),
        in_specs=[row_spec, parts_spec, full((hs, nint)), full((1, nint)),
                  full((nint, hs)), full((1, hs)), full((1, hs)),
                  full((1, hs)), full((hs, 4 * hs))],
        out_specs=[row_spec, xws_spec],
        out_shape=[jax.ShapeDtypeStruct((n, hs), f32),
                   jax.ShapeDtypeStruct((4, n, hs), f32)],
    )(hn0, parts0, params['ff_w1_0'], params['ff_b1_0'].reshape(1, -1),
      params['ff_w2_0'], params['ff_b2_0'].reshape(1, hs),
      params['ln_g_1'].reshape(1, hs), params['ln_b_1'].reshape(1, hs),
      wcat(1))

    parts1 = _sc_edge_scatter(xws1.reshape(4 * n, hs), gidx2, row2, n, hs)

    hw_row = params['head_w'].reshape(hs)[None, :]
    hf, vec, prop = pl.pallas_call(
        functools.partial(_final_body, nb, ngraphs),
        grid=(nb,),
        in_specs=[row_spec, parts_spec, full((hs, nint)), full((1, nint)),
                  full((nint, hs)), full((1, hs)), idx_spec,
                  full((1, hs)), full((1, 1))],
        out_specs=[row_spec,
                   pl.BlockSpec((ngraphs, hs), lambda i: (0, 0)),
                   pl.BlockSpec((ngraphs, 1), lambda i: (0, 0))],
        out_shape=[jax.ShapeDtypeStruct((n, hs), f32),
                   jax.ShapeDtypeStruct((ngraphs, hs), f32),
                   jax.ShapeDtypeStruct((ngraphs, 1), f32)],
        scratch_shapes=[pltpu.VMEM((1, ngraphs), f32)],
    )(hn1, parts1, params['ff_w1_1'], params['ff_b1_1'].reshape(1, -1),
      params['ff_w2_1'], params['ff_b2_1'].reshape(1, hs), batch3,
      hw_row, params['head_b'].reshape(1, 1))

    return prop, vec, hf
